# Initial kernel scaffold; baseline (speedup 1.0000x reference)
import jax
import jax.numpy as jnp
from jax.experimental import pallas as pl


def kernel(x, w_cd, b_cd, w_hd, b_hd, w_vd, b_vd, w_ad, b_ad, w_std, b_std, w_router, b_router, w1, b1, w2, b2):
    raise NotImplementedError("write your pallas kernel here")


# speedup vs baseline: 1.8636x; 1.8636x over previous
import functools

import jax
import jax.numpy as jnp
from jax.experimental import pallas as pl

_C = 96
_E = 5
_H = 224
_AD = (3, 0, 1, 6, 4, 2, 7, 8, 5)
_ROWS = 16


def _prep_kernel(cd_ref, ad_ref, hd_ref, vd_ref, std_ref, b_ref,
                 wmat_ref, bsum_ref):
    cd = cd_ref[...]
    ad = ad_ref[...]
    hd = hd_ref[...]
    vd = vd_ref[...]
    st = std_ref[...]
    s = jnp.sum(cd, axis=0)
    for t in range(9):
        kh, kw = divmod(t, 3)
        w = cd[t] + ad[t] - ad[_AD[t]] + st[t]
        if t == 4:
            w = w - s
        if kw == 0:
            w = w + hd[kh]
        elif kw == 2:
            w = w - hd[kh]
        if kh == 0:
            w = w + vd[kw]
        elif kh == 2:
            w = w - vd[kw]
        wmat_ref[t * _C:(t + 1) * _C, :] = w
    bsum_ref[...] = jnp.sum(b_ref[...], axis=0, keepdims=True)


def _fused_kernel(xp_ref, wmat_ref, bsum_ref, wr_ref, brt_ref,
                  w1s_ref, b1f_ref, w2s_ref, b2_ref, out_ref, *, rows):
    i = pl.program_id(0)
    t_cnt = rows * _H
    xs = xp_ref[pl.ds(i * rows, rows + 2)]
    pieces = []
    for kh in range(3):
        band = xs[kh:kh + rows]
        for kw in range(3):
            pieces.append(band[:, kw:kw + _H, :])
    cat = jnp.concatenate(pieces, axis=-1).reshape(t_cnt, 9 * _C)
    y = jnp.dot(cat, wmat_ref[...], preferred_element_type=jnp.float32)
    y = y + bsum_ref[...]

    logits = jnp.dot(y, wr_ref[...], preferred_element_type=jnp.float32)
    logits = logits + brt_ref[...]
    m = jnp.max(logits, axis=-1, keepdims=True)
    taken = jnp.zeros((t_cnt, 1), dtype=jnp.float32)
    sels = []
    for e in range(_E):
        eq = jnp.where(logits[:, e:e + 1] == m, 1.0, 0.0)
        s = eq * (1.0 - taken)
        taken = taken + s
        sels.append(s)

    h = jnp.dot(y, w1s_ref[...], preferred_element_type=jnp.float32)
    h = jnp.maximum(h + b1f_ref[...], 0.0)
    mask = jnp.concatenate(
        [jnp.broadcast_to(s, (t_cnt, _C)) for s in sels], axis=-1)
    h = h * mask
    o = jnp.dot(h, w2s_ref[...], preferred_element_type=jnp.float32)
    for e in range(_E):
        o = o + sels[e] * b2_ref[e]
    o = jnp.where(o >= 0.0, o, 0.01 * o)
    out_ref[...] = o.reshape(1, rows, _H, _C)


def kernel(x, w_cd, b_cd, w_hd, b_hd, w_vd, b_vd, w_ad, b_ad, w_std, b_std,
           w_router, b_router, w1, b1, w2, b2):
    C = _C
    cd9 = w_cd.reshape(C, C, 9).transpose(2, 1, 0)
    ad9 = w_ad.reshape(C, C, 9).transpose(2, 1, 0)
    std9 = w_std.reshape(C, C, 9).transpose(2, 1, 0)
    hd3 = w_hd.transpose(2, 1, 0)
    vd3 = w_vd.transpose(2, 1, 0)
    b5 = jnp.stack([b_cd, b_hd, b_vd, b_ad, b_std], axis=0)
    wmat, bsum = pl.pallas_call(
        _prep_kernel,
        out_shape=(jax.ShapeDtypeStruct((9 * C, C), jnp.float32),
                   jax.ShapeDtypeStruct((1, C), jnp.float32)),
    )(cd9, ad9, hd3, vd3, std9, b5)

    xp = jnp.pad(x[0], ((1, 1), (1, 1), (0, 0)))
    w1s = w1.transpose(1, 0, 2).reshape(C, _E * C)
    b1f = b1.reshape(1, _E * C)
    w2s = w2.reshape(_E * C, C)
    brt = b_router.reshape(1, _E)

    rows = _ROWS
    grid = _H // rows
    out = pl.pallas_call(
        functools.partial(_fused_kernel, rows=rows),
        grid=(grid,),
        in_specs=[
            pl.BlockSpec((_H + 2, _H + 2, C), lambda i: (0, 0, 0)),
            pl.BlockSpec((9 * C, C), lambda i: (0, 0)),
            pl.BlockSpec((1, C), lambda i: (0, 0)),
            pl.BlockSpec((C, _E), lambda i: (0, 0)),
            pl.BlockSpec((1, _E), lambda i: (0, 0)),
            pl.BlockSpec((C, _E * C), lambda i: (0, 0)),
            pl.BlockSpec((1, _E * C), lambda i: (0, 0)),
            pl.BlockSpec((_E * C, C), lambda i: (0, 0)),
            pl.BlockSpec((_E, C), lambda i: (0, 0)),
        ],
        out_specs=pl.BlockSpec((1, rows, _H, C), lambda i: (0, i, 0, 0)),
        out_shape=jax.ShapeDtypeStruct((1, _H, _H, C), jnp.float32),
    )(xp, wmat, bsum, w_router, brt, w1s, b1f, w2s, b2)
    return out


# speedup vs baseline: 2.2548x; 1.2099x over previous
import functools

import jax
import jax.numpy as jnp
import numpy as np
from jax.experimental import pallas as pl

_C = 96
_E = 5
_H = 224
_AD = (3, 0, 1, 6, 4, 2, 7, 8, 5)
_ROWS = 16


def _prep_kernel(cd_ref, ad_ref, hd_ref, vd_ref, std_ref, b_ref,
                 wr_ref, brt_ref, wext_ref, bext_ref):
    cd = cd_ref[...]
    ad = ad_ref[...]
    hd = hd_ref[...]
    vd = vd_ref[...]
    st = std_ref[...]
    s = jnp.sum(cd, axis=0)
    taps = []
    for t in range(9):
        kh, kw = divmod(t, 3)
        w = cd[t] + ad[t] - ad[_AD[t]] + st[t]
        if t == 4:
            w = w - s
        if kw == 0:
            w = w + hd[kh]
        elif kw == 2:
            w = w - hd[kh]
        if kh == 0:
            w = w + vd[kw]
        elif kh == 2:
            w = w - vd[kw]
        taps.append(w)
    wmat = jnp.concatenate(taps, axis=0)
    wr2 = jnp.dot(wmat, wr_ref[...], preferred_element_type=jnp.float32)
    wext_ref[...] = jnp.concatenate([wmat, wr2], axis=-1)
    bsum = jnp.sum(b_ref[...], axis=0, keepdims=True)
    blog = jnp.dot(bsum, wr_ref[...],
                   preferred_element_type=jnp.float32) + brt_ref[...]
    bext_ref[...] = jnp.concatenate([bsum, blog], axis=-1)


def _fused_kernel(xp_ref, wext_ref, bext_ref, tri_ref, pmask_ref,
                  w1s_ref, b1f_ref, w2s_ref, b2_ref, out_ref, *, rows):
    i = pl.program_id(0)
    t_cnt = rows * _H
    xs = xp_ref[pl.ds(i * rows, rows + 2)]
    pieces = []
    for kh in range(3):
        band = xs[kh:kh + rows]
        for kw in range(3):
            pieces.append(band[:, kw:kw + _H, :])
    cat = jnp.concatenate(pieces, axis=-1).reshape(t_cnt, 9 * _C)
    yl = jnp.dot(cat, wext_ref[...], preferred_element_type=jnp.float32)
    yl = yl + bext_ref[...]
    y = yl[:, :_C]
    logits = yl[:, _C:]

    m = jnp.max(logits, axis=-1, keepdims=True)
    eq = jnp.where(logits == m, 1.0, 0.0)
    csum = jnp.dot(eq, tri_ref[...], preferred_element_type=jnp.float32)
    sel = eq * jnp.where(csum == 1.0, 1.0, 0.0)
    mask = jnp.dot(sel, pmask_ref[...], preferred_element_type=jnp.float32)
    bias2 = jnp.dot(sel, b2_ref[...], preferred_element_type=jnp.float32)

    h = jnp.dot(y, w1s_ref[...], preferred_element_type=jnp.float32)
    h = jnp.maximum(h + b1f_ref[...], 0.0) * mask
    o = jnp.dot(h, w2s_ref[...], preferred_element_type=jnp.float32)
    o = o + bias2
    o = jnp.where(o >= 0.0, o, 0.01 * o)
    out_ref[...] = o.reshape(1, rows, _H, _C)


def kernel(x, w_cd, b_cd, w_hd, b_hd, w_vd, b_vd, w_ad, b_ad, w_std, b_std,
           w_router, b_router, w1, b1, w2, b2):
    C = _C
    cd9 = w_cd.reshape(C, C, 9).transpose(2, 1, 0)
    ad9 = w_ad.reshape(C, C, 9).transpose(2, 1, 0)
    std9 = w_std.reshape(C, C, 9).transpose(2, 1, 0)
    hd3 = w_hd.transpose(2, 1, 0)
    vd3 = w_vd.transpose(2, 1, 0)
    b5 = jnp.stack([b_cd, b_hd, b_vd, b_ad, b_std], axis=0)
    brt = b_router.reshape(1, _E)
    wext, bext = pl.pallas_call(
        _prep_kernel,
        out_shape=(jax.ShapeDtypeStruct((9 * C, C + _E), jnp.float32),
                   jax.ShapeDtypeStruct((1, C + _E), jnp.float32)),
    )(cd9, ad9, hd3, vd3, std9, b5, w_router, brt)

    xp = jnp.pad(x[0], ((1, 1), (1, 1), (0, 0)))
    w1s = w1.transpose(1, 0, 2).reshape(C, _E * C)
    b1f = b1.reshape(1, _E * C)
    w2s = w2.reshape(_E * C, C)
    tri = jnp.asarray(np.triu(np.ones((_E, _E), np.float32)))
    pmask = jnp.asarray(np.kron(np.eye(_E, dtype=np.float32),
                                np.ones((1, C), np.float32)))

    rows = _ROWS
    grid = _H // rows
    out = pl.pallas_call(
        functools.partial(_fused_kernel, rows=rows),
        grid=(grid,),
        in_specs=[
            pl.BlockSpec((_H + 2, _H + 2, C), lambda i: (0, 0, 0)),
            pl.BlockSpec((9 * C, C + _E), lambda i: (0, 0)),
            pl.BlockSpec((1, C + _E), lambda i: (0, 0)),
            pl.BlockSpec((_E, _E), lambda i: (0, 0)),
            pl.BlockSpec((_E, _E * C), lambda i: (0, 0)),
            pl.BlockSpec((C, _E * C), lambda i: (0, 0)),
            pl.BlockSpec((1, _E * C), lambda i: (0, 0)),
            pl.BlockSpec((_E * C, C), lambda i: (0, 0)),
            pl.BlockSpec((_E, C), lambda i: (0, 0)),
        ],
        out_specs=pl.BlockSpec((1, rows, _H, C), lambda i: (0, i, 0, 0)),
        out_shape=jax.ShapeDtypeStruct((1, _H, _H, C), jnp.float32),
    )(xp, wext, bext, tri, pmask, w1s, b1f, w2s, b2)
    return out


# speedup vs baseline: 2.5696x; 1.1396x over previous
import functools

import jax
import jax.numpy as jnp
import numpy as np
from jax.experimental import pallas as pl

_C = 96
_E = 5
_H = 224
_AD = (3, 0, 1, 6, 4, 2, 7, 8, 5)
_ROWS = 16


def _prep_kernel(cd_ref, ad_ref, hd_ref, vd_ref, std_ref, b_ref,
                 wext_ref, bext_ref):
    cd = cd_ref[...]
    ad = ad_ref[...]
    hd = hd_ref[...]
    vd = vd_ref[...]
    st = std_ref[...]
    s = jnp.sum(cd, axis=0)
    taps = []
    for t in range(9):
        kh, kw = divmod(t, 3)
        w = cd[t] + ad[t] - ad[_AD[t]] + st[t]
        if t == 4:
            w = w - s
        if kw == 0:
            w = w + hd[kh]
        elif kw == 2:
            w = w - hd[kh]
        if kh == 0:
            w = w + vd[kw]
        elif kh == 2:
            w = w - vd[kw]
        taps.append(w)
    wext_ref[...] = jnp.concatenate(taps, axis=0)
    bext_ref[...] = jnp.sum(b_ref[...], axis=0, keepdims=True)


def _fused_kernel(xp_ref, wext_ref, bext_ref, wr_ref, brt_ref, tri_ref,
                  pmask_ref, w1s_ref, b1f_ref, w2s_ref, b2_ref, out_ref, *,
                  rows):
    i = pl.program_id(0)
    t_cnt = rows * _H
    xs = xp_ref[pl.ds(i * rows, rows + 2)]
    pieces = []
    for kh in range(3):
        band = xs[kh:kh + rows]
        for kw in range(3):
            pieces.append(band[:, kw:kw + _H, :])
    cat = jnp.concatenate(pieces, axis=-1).reshape(t_cnt, 9 * _C)
    y = jnp.dot(cat, wext_ref[...], preferred_element_type=jnp.float32)
    y = y + bext_ref[...]
    logits = jnp.dot(y, wr_ref[...], preferred_element_type=jnp.float32)
    logits = logits + brt_ref[...]

    m = jnp.max(logits, axis=-1, keepdims=True)
    eq = jnp.where(logits == m, 1.0, 0.0)
    csum = jnp.dot(eq, tri_ref[...], preferred_element_type=jnp.float32)
    sel = eq * jnp.where(csum == 1.0, 1.0, 0.0)
    mask = jnp.dot(sel, pmask_ref[...], preferred_element_type=jnp.float32)
    bias2 = jnp.dot(sel, b2_ref[...], preferred_element_type=jnp.float32)

    h = jnp.dot(y, w1s_ref[...], preferred_element_type=jnp.float32)
    h = jnp.maximum(h + b1f_ref[...], 0.0) * mask
    o = jnp.dot(h, w2s_ref[...], preferred_element_type=jnp.float32)
    o = o + bias2
    o = jnp.where(o >= 0.0, o, 0.01 * o)
    out_ref[...] = o.reshape(1, rows, _H, _C)


def kernel(x, w_cd, b_cd, w_hd, b_hd, w_vd, b_vd, w_ad, b_ad, w_std, b_std,
           w_router, b_router, w1, b1, w2, b2):
    C = _C
    cd9 = w_cd.reshape(C, C, 9).transpose(2, 1, 0)
    ad9 = w_ad.reshape(C, C, 9).transpose(2, 1, 0)
    std9 = w_std.reshape(C, C, 9).transpose(2, 1, 0)
    hd3 = w_hd.transpose(2, 1, 0)
    vd3 = w_vd.transpose(2, 1, 0)
    b5 = jnp.stack([b_cd, b_hd, b_vd, b_ad, b_std], axis=0)
    brt = b_router.reshape(1, _E)
    wext, bext = pl.pallas_call(
        _prep_kernel,
        out_shape=(jax.ShapeDtypeStruct((9 * C, C), jnp.float32),
                   jax.ShapeDtypeStruct((1, C), jnp.float32)),
    )(cd9, ad9, hd3, vd3, std9, b5)

    xp = jnp.pad(x[0], ((1, 1), (1, 1), (0, 0)))
    w1s = w1.transpose(1, 0, 2).reshape(C, _E * C)
    b1f = b1.reshape(1, _E * C)
    w2s = w2.reshape(_E * C, C)
    tri = jnp.asarray(np.triu(np.ones((_E, _E), np.float32)))
    pmask = jnp.asarray(np.kron(np.eye(_E, dtype=np.float32),
                                np.ones((1, C), np.float32)))

    rows = _ROWS
    grid = _H // rows
    out = pl.pallas_call(
        functools.partial(_fused_kernel, rows=rows),
        grid=(grid,),
        in_specs=[
            pl.BlockSpec((_H + 2, _H + 2, C), lambda i: (0, 0, 0)),
            pl.BlockSpec((9 * C, C), lambda i: (0, 0)),
            pl.BlockSpec((1, C), lambda i: (0, 0)),
            pl.BlockSpec((C, _E), lambda i: (0, 0)),
            pl.BlockSpec((1, _E), lambda i: (0, 0)),
            pl.BlockSpec((_E, _E), lambda i: (0, 0)),
            pl.BlockSpec((_E, _E * C), lambda i: (0, 0)),
            pl.BlockSpec((C, _E * C), lambda i: (0, 0)),
            pl.BlockSpec((1, _E * C), lambda i: (0, 0)),
            pl.BlockSpec((_E * C, C), lambda i: (0, 0)),
            pl.BlockSpec((_E, C), lambda i: (0, 0)),
        ],
        out_specs=pl.BlockSpec((1, rows, _H, C), lambda i: (0, i, 0, 0)),
        out_shape=jax.ShapeDtypeStruct((1, _H, _H, C), jnp.float32),
    )(xp, wext, bext, w_router, brt, tri, pmask, w1s, b1f, w2s, b2)
    return out


# speedup vs baseline: 3.3463x; 1.3023x over previous
import functools

import jax
import jax.numpy as jnp
import numpy as np
from jax.experimental import pallas as pl

_C = 96
_E = 5
_H = 224
_AD = (3, 0, 1, 6, 4, 2, 7, 8, 5)
_ROWS = 16


def _prep_kernel(cd_ref, ad_ref, hd_ref, vd_ref, std_ref, b_ref,
                 wext_ref, bext_ref):
    cd = cd_ref[...]
    ad = ad_ref[...]
    hd = hd_ref[...]
    vd = vd_ref[...]
    st = std_ref[...]
    s = jnp.sum(cd, axis=0)
    taps = []
    for t in range(9):
        kh, kw = divmod(t, 3)
        w = cd[t] + ad[t] - ad[_AD[t]] + st[t]
        if t == 4:
            w = w - s
        if kw == 0:
            w = w + hd[kh]
        elif kw == 2:
            w = w - hd[kh]
        if kh == 0:
            w = w + vd[kw]
        elif kh == 2:
            w = w - vd[kw]
        taps.append(w)
    wext_ref[...] = jnp.concatenate(taps, axis=0)
    bext_ref[...] = jnp.sum(b_ref[...], axis=0, keepdims=True)


def _pad_kernel(x_ref, xp_ref):
    z_row = jnp.zeros((1, _H + 2, _C), dtype=jnp.float32)
    z_col = jnp.zeros((_H, 1, _C), dtype=jnp.float32)
    xp_ref[0:1] = z_row
    xp_ref[_H + 1:_H + 2] = z_row
    xp_ref[1:_H + 1] = jnp.concatenate([z_col, x_ref[0], z_col], axis=1)


def _fused_kernel(xp_ref, wext_ref, bext_ref, wr_ref, brt_ref, tri_ref,
                  pmask_ref, w1s_ref, b1f_ref, w2s_ref, b2_ref, out_ref, *,
                  rows):
    i = pl.program_id(0)
    t_cnt = rows * _H
    xs = xp_ref[pl.ds(i * rows, rows + 2)]
    pieces = []
    for kh in range(3):
        band = xs[kh:kh + rows]
        for kw in range(3):
            pieces.append(band[:, kw:kw + _H, :])
    cat = jnp.concatenate(pieces, axis=-1).reshape(t_cnt, 9 * _C)
    y = jnp.dot(cat, wext_ref[...], preferred_element_type=jnp.float32)
    y = y + bext_ref[...]
    logits = jnp.dot(y, wr_ref[...], preferred_element_type=jnp.float32)
    logits = logits + brt_ref[...]

    m = jnp.max(logits, axis=-1, keepdims=True)
    eq = jnp.where(logits == m, 1.0, 0.0)
    csum = jnp.dot(eq, tri_ref[...], preferred_element_type=jnp.float32)
    sel = eq * jnp.where(csum == 1.0, 1.0, 0.0)
    mask = jnp.dot(sel, pmask_ref[...], preferred_element_type=jnp.float32)
    bias2 = jnp.dot(sel, b2_ref[...], preferred_element_type=jnp.float32)

    h = jnp.dot(y, w1s_ref[...], preferred_element_type=jnp.float32)
    h = jnp.maximum(h + b1f_ref[...], 0.0) * mask
    o = jnp.dot(h, w2s_ref[...], preferred_element_type=jnp.float32)
    o = o + bias2
    o = jnp.where(o >= 0.0, o, 0.01 * o)
    out_ref[...] = o.reshape(1, rows, _H, _C)


def kernel(x, w_cd, b_cd, w_hd, b_hd, w_vd, b_vd, w_ad, b_ad, w_std, b_std,
           w_router, b_router, w1, b1, w2, b2):
    C = _C
    cd9 = w_cd.reshape(C, C, 9).transpose(2, 1, 0)
    ad9 = w_ad.reshape(C, C, 9).transpose(2, 1, 0)
    std9 = w_std.reshape(C, C, 9).transpose(2, 1, 0)
    hd3 = w_hd.transpose(2, 1, 0)
    vd3 = w_vd.transpose(2, 1, 0)
    b5 = jnp.stack([b_cd, b_hd, b_vd, b_ad, b_std], axis=0)
    brt = b_router.reshape(1, _E)
    wext, bext = pl.pallas_call(
        _prep_kernel,
        out_shape=(jax.ShapeDtypeStruct((9 * C, C), jnp.float32),
                   jax.ShapeDtypeStruct((1, C), jnp.float32)),
    )(cd9, ad9, hd3, vd3, std9, b5)

    xp = pl.pallas_call(
        _pad_kernel,
        out_shape=jax.ShapeDtypeStruct((_H + 2, _H + 2, C), jnp.float32),
    )(x)
    w1s = w1.transpose(1, 0, 2).reshape(C, _E * C)
    b1f = b1.reshape(1, _E * C)
    w2s = w2.reshape(_E * C, C)
    tri = jnp.asarray(np.triu(np.ones((_E, _E), np.float32)))
    pmask = jnp.asarray(np.kron(np.eye(_E, dtype=np.float32),
                                np.ones((1, C), np.float32)))

    rows = _ROWS
    grid = _H // rows
    out = pl.pallas_call(
        functools.partial(_fused_kernel, rows=rows),
        grid=(grid,),
        in_specs=[
            pl.BlockSpec((_H + 2, _H + 2, C), lambda i: (0, 0, 0)),
            pl.BlockSpec((9 * C, C), lambda i: (0, 0)),
            pl.BlockSpec((1, C), lambda i: (0, 0)),
            pl.BlockSpec((C, _E), lambda i: (0, 0)),
            pl.BlockSpec((1, _E), lambda i: (0, 0)),
            pl.BlockSpec((_E, _E), lambda i: (0, 0)),
            pl.BlockSpec((_E, _E * C), lambda i: (0, 0)),
            pl.BlockSpec((C, _E * C), lambda i: (0, 0)),
            pl.BlockSpec((1, _E * C), lambda i: (0, 0)),
            pl.BlockSpec((_E * C, C), lambda i: (0, 0)),
            pl.BlockSpec((_E, C), lambda i: (0, 0)),
        ],
        out_specs=pl.BlockSpec((1, rows, _H, C), lambda i: (0, i, 0, 0)),
        out_shape=jax.ShapeDtypeStruct((1, _H, _H, C), jnp.float32),
    )(xp, wext, bext, w_router, brt, tri, pmask, w1s, b1f, w2s, b2)
    return out


# speedup vs baseline: 3.8006x; 1.1358x over previous
import functools

import jax
import jax.numpy as jnp
import numpy as np
from jax.experimental import pallas as pl
from jax.experimental.pallas import tpu as pltpu

_C = 96
_E = 5
_H = 224
_AD = (3, 0, 1, 6, 4, 2, 7, 8, 5)
_ROWS = 16


def _prep_kernel(cd_ref, ad_ref, hd_ref, vd_ref, std_ref, b_ref,
                 wext_ref, bext_ref):
    cd = cd_ref[...]
    ad = ad_ref[...]
    hd = hd_ref[...]
    vd = vd_ref[...]
    st = std_ref[...]
    s = jnp.sum(cd, axis=0)
    taps = []
    for t in range(9):
        kh, kw = divmod(t, 3)
        w = cd[t] + ad[t] - ad[_AD[t]] + st[t]
        if t == 4:
            w = w - s
        if kw == 0:
            w = w + hd[kh]
        elif kw == 2:
            w = w - hd[kh]
        if kh == 0:
            w = w + vd[kw]
        elif kh == 2:
            w = w - vd[kw]
        taps.append(w)
    wext_ref[...] = jnp.concatenate(taps, axis=0)
    bext_ref[...] = jnp.sum(b_ref[...], axis=0, keepdims=True)


def _fused_kernel(x_ref, wext_ref, bext_ref, wr_ref, brt_ref, tri_ref,
                  pmask_ref, w1s_ref, b1f_ref, w2s_ref, b2_ref, out_ref,
                  xp_ref, *, rows):
    i = pl.program_id(0)
    t_cnt = rows * _H
    n_blk = _H // rows

    @pl.when(i == 0)
    def _init_borders():
        xp_ref[0:1] = jnp.zeros((1, _H + 2, _C), dtype=jnp.float32)
        xp_ref[_H + 1:_H + 2] = jnp.zeros((1, _H + 2, _C), dtype=jnp.float32)
        xp_ref[:, 0:1, :] = jnp.zeros((_H + 2, 1, _C), dtype=jnp.float32)
        xp_ref[:, _H + 1:_H + 2, :] = jnp.zeros((_H + 2, 1, _C),
                                                dtype=jnp.float32)

    @pl.when(i < n_blk)
    def _stage_rows():
        xp_ref[pl.ds(1 + i * rows, rows), 1:_H + 1, :] = x_ref[0]

    @pl.when(i > 0)
    def _compute():
        j = i - 1
        xs = xp_ref[pl.ds(j * rows, rows + 2)]
        pieces = []
        for kh in range(3):
            band = xs[kh:kh + rows]
            for kw in range(3):
                pieces.append(band[:, kw:kw + _H, :])
        cat = jnp.concatenate(pieces, axis=-1).reshape(t_cnt, 9 * _C)
        y = jnp.dot(cat, wext_ref[...], preferred_element_type=jnp.float32)
        y = y + bext_ref[...]
        logits = jnp.dot(y, wr_ref[...], preferred_element_type=jnp.float32)
        logits = logits + brt_ref[...]

        m = jnp.max(logits, axis=-1, keepdims=True)
        eq = jnp.where(logits == m, 1.0, 0.0)
        csum = jnp.dot(eq, tri_ref[...], preferred_element_type=jnp.float32)
        sel = eq * jnp.where(csum == 1.0, 1.0, 0.0)
        mask = jnp.dot(sel, pmask_ref[...],
                       preferred_element_type=jnp.float32)
        bias2 = jnp.dot(sel, b2_ref[...], preferred_element_type=jnp.float32)

        h = jnp.dot(y, w1s_ref[...], preferred_element_type=jnp.float32)
        h = jnp.maximum(h + b1f_ref[...], 0.0) * mask
        o = jnp.dot(h, w2s_ref[...], preferred_element_type=jnp.float32)
        o = o + bias2
        o = jnp.where(o >= 0.0, o, 0.01 * o)
        out_ref[...] = o.reshape(1, rows, _H, _C)


def kernel(x, w_cd, b_cd, w_hd, b_hd, w_vd, b_vd, w_ad, b_ad, w_std, b_std,
           w_router, b_router, w1, b1, w2, b2):
    C = _C
    cd9 = w_cd.reshape(C, C, 9).transpose(2, 1, 0)
    ad9 = w_ad.reshape(C, C, 9).transpose(2, 1, 0)
    std9 = w_std.reshape(C, C, 9).transpose(2, 1, 0)
    hd3 = w_hd.transpose(2, 1, 0)
    vd3 = w_vd.transpose(2, 1, 0)
    b5 = jnp.stack([b_cd, b_hd, b_vd, b_ad, b_std], axis=0)
    brt = b_router.reshape(1, _E)
    wext, bext = pl.pallas_call(
        _prep_kernel,
        out_shape=(jax.ShapeDtypeStruct((9 * C, C), jnp.float32),
                   jax.ShapeDtypeStruct((1, C), jnp.float32)),
    )(cd9, ad9, hd3, vd3, std9, b5)

    w1s = w1.transpose(1, 0, 2).reshape(C, _E * C)
    b1f = b1.reshape(1, _E * C)
    w2s = w2.reshape(_E * C, C)
    tri = jnp.asarray(np.triu(np.ones((_E, _E), np.float32)))
    pmask = jnp.asarray(np.kron(np.eye(_E, dtype=np.float32),
                                np.ones((1, C), np.float32)))

    rows = _ROWS
    n_blk = _H // rows
    out = pl.pallas_call(
        functools.partial(_fused_kernel, rows=rows),
        grid=(n_blk + 1,),
        in_specs=[
            pl.BlockSpec((1, rows, _H, C),
                         lambda i: (0, jnp.minimum(i, _H // _ROWS - 1), 0, 0)),
            pl.BlockSpec((9 * C, C), lambda i: (0, 0)),
            pl.BlockSpec((1, C), lambda i: (0, 0)),
            pl.BlockSpec((C, _E), lambda i: (0, 0)),
            pl.BlockSpec((1, _E), lambda i: (0, 0)),
            pl.BlockSpec((_E, _E), lambda i: (0, 0)),
            pl.BlockSpec((_E, _E * C), lambda i: (0, 0)),
            pl.BlockSpec((C, _E * C), lambda i: (0, 0)),
            pl.BlockSpec((1, _E * C), lambda i: (0, 0)),
            pl.BlockSpec((_E * C, C), lambda i: (0, 0)),
            pl.BlockSpec((_E, C), lambda i: (0, 0)),
        ],
        out_specs=pl.BlockSpec((1, rows, _H, C),
                               lambda i: (0, jnp.maximum(i - 1, 0), 0, 0)),
        out_shape=jax.ShapeDtypeStruct((1, _H, _H, C), jnp.float32),
        scratch_shapes=[pltpu.VMEM((_H + 2, _H + 2, C), jnp.float32)],
        compiler_params=pltpu.CompilerParams(
            vmem_limit_bytes=100 * 1024 * 1024),
    )(x, wext, bext, w_router, brt, tri, pmask, w1s, b1f, w2s, b2)
    return out
